# baseline (speedup 1.0000x reference)
import functools

import jax
import jax.numpy as jnp
from jax import lax
from jax.experimental import pallas as pl
from jax.experimental.pallas import tpu as pltpu
from jax.experimental.pallas import tpu_sc as plsc

N = 320000
D = 128
NUM_CHUNKS = 8
NUM_SEG = 10000
CHUNK = N // NUM_CHUNKS
OUT_PER_CHUNK = NUM_SEG - 1

_INFO = plsc.get_sparse_core_info()
NC = _INFO.num_cores
NS = _INFO.num_subcores
WORKERS = NC * NS
PARTS = WORKERS // NUM_CHUNKS
SEG_PER_PART = NUM_SEG // PARTS
TILE_SEGS = 250
SUBTASKS = SEG_PER_PART // TILE_SEGS
BLK = 128
NVEC = D // 16


def _build():
    mesh = plsc.VectorSubcoreMesh(core_axis_name="c", subcore_axis_name="s")

    @functools.partial(
        pl.kernel,
        mesh=mesh,
        out_type=jax.ShapeDtypeStruct((NUM_CHUNKS * OUT_PER_CHUNK, D),
                                      jnp.float32),
        compiler_params=pltpu.CompilerParams(use_tc_tiling_on_sc=False),
        scratch_types=[
            pltpu.VMEM((CHUNK + 16,), jnp.int32),
            pltpu.VMEM((TILE_SEGS, D), jnp.float32),
            pltpu.VMEM((TILE_SEGS, 16), jnp.float32),
            pltpu.VMEM((BLK, D), jnp.float32),
        ],
    )
    def pooled(x_hbm, ids_hbm, out_hbm, ids_v, acc_v, cnt_v, xbuf_v):
        wid = lax.axis_index("s") * NC + lax.axis_index("c")
        chunk = wid // PARTS
        part = wid % PARTS

        pltpu.sync_copy(ids_hbm.at[pl.ds(chunk * CHUNK, CHUNK)],
                        ids_v.at[pl.ds(0, CHUNK)])

        def idat(i):
            return ids_v[pl.ds(i, 16)][0]

        def ssorted(t):
            def body(_, st):
                lo, hi = st
                mid = (lo + hi) // 2
                v = idat(mid)
                lo2 = jnp.where(v < t, mid + 1, lo)
                hi2 = jnp.where(v < t, hi, mid)
                done = lo >= hi
                return (jnp.where(done, lo, lo2), jnp.where(done, hi, hi2))

            return lax.fori_loop(0, 16, body,
                                 (jnp.int32(0), jnp.int32(CHUNK)))[0]

        zero16 = jnp.zeros((16,), jnp.float32)
        one16 = jnp.ones((16,), jnp.float32)

        def qbody(q, _):
            lo_seg = part * SEG_PER_PART + q * TILE_SEGS
            rs = ssorted(lo_seg)
            re = ssorted(lo_seg + TILE_SEGS)

            def zbody(s, _):
                for v in range(NVEC):
                    acc_v[s, pl.ds(v * 16, 16)] = zero16
                cnt_v[s, :] = zero16
                return 0

            lax.fori_loop(0, TILE_SEGS, zbody, 0)

            kb0 = rs // BLK
            kb1 = (re + BLK - 1) // BLK

            def bbody(k, _):
                pltpu.sync_copy(
                    x_hbm.at[pl.ds(chunk * CHUNK + k * BLK, BLK)], xbuf_v)
                i0 = jnp.maximum(rs, k * BLK)
                i1 = jnp.minimum(re, (k + 1) * BLK)

                def rbody(r, _):
                    lid = idat(r) - lo_seg
                    ii = r - k * BLK
                    for v in range(NVEC):
                        sl = pl.ds(v * 16, 16)
                        plsc.addupdate(acc_v.at[lid, sl], xbuf_v[ii, sl])
                    plsc.addupdate(cnt_v.at[lid], one16)
                    return 0

                lax.fori_loop(i0, i1, rbody, 0)
                return 0

            lax.fori_loop(kb0, kb1, bbody, 0)

            def fbody(s, _):
                scale = 1.0 / jnp.maximum(cnt_v[s, :], 1.0)
                for v in range(NVEC):
                    sl = pl.ds(v * 16, 16)
                    acc_v[s, sl] = acc_v[s, sl] * scale
                return 0

            lax.fori_loop(0, TILE_SEGS, fbody, 0)

            out0 = chunk * OUT_PER_CHUNK

            @pl.when(lo_seg == 0)
            def _():
                pltpu.sync_copy(
                    acc_v.at[pl.ds(1, TILE_SEGS - 1)],
                    out_hbm.at[pl.ds(out0, TILE_SEGS - 1)])

            @pl.when(lo_seg > 0)
            def _():
                pltpu.sync_copy(
                    acc_v.at[pl.ds(0, TILE_SEGS)],
                    out_hbm.at[pl.ds(out0 + lo_seg - 1, TILE_SEGS)])

            return 0

        lax.fori_loop(0, SUBTASKS, qbody, 0)

    return pooled


_POOLED = _build()


@jax.jit
def _run(x, batch_actor):
    return _POOLED(x, batch_actor)


def kernel(x, chunk_sizes, batch_actor):
    del chunk_sizes
    return _run(x, batch_actor)


# speedup vs baseline: 1.6082x; 1.6082x over previous
import functools

import jax
import jax.numpy as jnp
from jax import lax
from jax.experimental import pallas as pl
from jax.experimental.pallas import tpu as pltpu
from jax.experimental.pallas import tpu_sc as plsc

N = 320000
D = 128
NUM_CHUNKS = 8
NUM_SEG = 10000
CHUNK = N // NUM_CHUNKS
OUT_PER_CHUNK = NUM_SEG - 1

_INFO = plsc.get_sparse_core_info()
NC = _INFO.num_cores
NS = _INFO.num_subcores
WORKERS = NC * NS
PARTS = WORKERS // NUM_CHUNKS
SEG_PER_PART = NUM_SEG // PARTS
TILE_SEGS = 250
SUBTASKS = SEG_PER_PART // TILE_SEGS
BLK = 128
NVEC = D // 16


def _build():
    mesh = plsc.VectorSubcoreMesh(core_axis_name="c", subcore_axis_name="s")

    @functools.partial(
        pl.kernel,
        mesh=mesh,
        out_type=jax.ShapeDtypeStruct((NUM_CHUNKS * OUT_PER_CHUNK, D),
                                      jnp.float32),
        compiler_params=pltpu.CompilerParams(use_tc_tiling_on_sc=False),
        scratch_types=[
            pltpu.VMEM((CHUNK + 16,), jnp.int32),
            pltpu.VMEM((TILE_SEGS, D), jnp.float32),
            pltpu.VMEM((TILE_SEGS, 16), jnp.float32),
            pltpu.VMEM((BLK, D), jnp.float32),
            pltpu.VMEM((BLK, D), jnp.float32),
            pltpu.SemaphoreType.DMA,
            pltpu.SemaphoreType.DMA,
        ],
    )
    def pooled(x_hbm, ids_hbm, out_hbm, ids_v, acc_v, cnt_v, xa_v, xb_v,
               sema, semb):
        wid = lax.axis_index("s") * NC + lax.axis_index("c")
        chunk = wid // PARTS
        part = wid % PARTS

        pltpu.sync_copy(ids_hbm.at[pl.ds(chunk * CHUNK, CHUNK)],
                        ids_v.at[pl.ds(0, CHUNK)])

        def idat(i):
            return ids_v[pl.ds(i, 16)][0]

        def ssorted(t):
            def body(_, st):
                lo, hi = st
                mid = (lo + hi) // 2
                v = idat(mid)
                lo2 = jnp.where(v < t, mid + 1, lo)
                hi2 = jnp.where(v < t, hi, mid)
                done = lo >= hi
                return (jnp.where(done, lo, lo2), jnp.where(done, hi, hi2))

            return lax.fori_loop(0, 16, body,
                                 (jnp.int32(0), jnp.int32(CHUNK)))[0]

        zero16 = jnp.zeros((16,), jnp.float32)
        one16 = jnp.ones((16,), jnp.float32)

        def qbody(q, _):
            lo_seg = part * SEG_PER_PART + q * TILE_SEGS
            rs = ssorted(lo_seg)
            re = ssorted(lo_seg + TILE_SEGS)

            def zbody(s, _):
                for v in range(NVEC):
                    acc_v[s, pl.ds(v * 16, 16)] = zero16
                cnt_v[s, :] = zero16
                return 0

            lax.fori_loop(0, TILE_SEGS, zbody, 0)

            kb0 = rs // BLK
            kb1 = (re + BLK - 1) // BLK
            nb = kb1 - kb0

            def issue(k, buf, sem):
                pltpu.make_async_copy(
                    x_hbm.at[pl.ds(chunk * CHUNK + k * BLK, BLK)],
                    buf, sem).start()

            def wait(buf, sem):
                pltpu.make_async_copy(
                    x_hbm.at[pl.ds(0, BLK)], buf, sem).wait()

            def row_add(buf, lid, ii):
                for v in range(NVEC):
                    sl = pl.ds(v * 16, 16)
                    plsc.addupdate(acc_v.at[lid, sl], buf[ii, sl])
                plsc.addupdate(cnt_v.at[lid], one16)

            def process(k, buf):
                def gbody(g, _):
                    rb = k * BLK + g * 16
                    i0g = jnp.maximum(rs, rb)
                    i1g = jnp.minimum(re, rb + 16)

                    @pl.when(i1g - i0g == 16)
                    def _():
                        ids16 = ids_v[pl.ds(rb, 16)]
                        for j in range(16):
                            row_add(buf, ids16[j] - lo_seg, g * 16 + j)

                    @pl.when((i1g - i0g < 16) & (i1g > i0g))
                    def _():
                        def rbody(r, _):
                            row_add(buf, idat(r) - lo_seg, r - k * BLK)
                            return 0

                        lax.fori_loop(i0g, i1g, rbody, 0)

                    return 0

                lax.fori_loop(0, BLK // 16, gbody, 0)

            @pl.when(nb > 0)
            def _():
                issue(kb0, xa_v, sema)

            def tbody(t2, _):
                ke = kb0 + 2 * t2
                ko = ke + 1

                @pl.when(ke < kb1)
                def _():
                    @pl.when(ko < kb1)
                    def _():
                        issue(ko, xb_v, semb)

                    wait(xa_v, sema)
                    process(ke, xa_v)

                @pl.when(ko < kb1)
                def _():
                    @pl.when(ko + 1 < kb1)
                    def _():
                        issue(ko + 1, xa_v, sema)

                    wait(xb_v, semb)
                    process(ko, xb_v)

                return 0

            lax.fori_loop(0, (nb + 1) // 2, tbody, 0)

            def fbody(s, _):
                scale = 1.0 / jnp.maximum(cnt_v[s, :], 1.0)
                for v in range(NVEC):
                    sl = pl.ds(v * 16, 16)
                    acc_v[s, sl] = acc_v[s, sl] * scale
                return 0

            lax.fori_loop(0, TILE_SEGS, fbody, 0)

            out0 = chunk * OUT_PER_CHUNK

            @pl.when(lo_seg == 0)
            def _():
                pltpu.sync_copy(
                    acc_v.at[pl.ds(1, TILE_SEGS - 1)],
                    out_hbm.at[pl.ds(out0, TILE_SEGS - 1)])

            @pl.when(lo_seg > 0)
            def _():
                pltpu.sync_copy(
                    acc_v.at[pl.ds(0, TILE_SEGS)],
                    out_hbm.at[pl.ds(out0 + lo_seg - 1, TILE_SEGS)])

            return 0

        lax.fori_loop(0, SUBTASKS, qbody, 0)

    return pooled


_POOLED = _build()


@jax.jit
def _run(x, batch_actor):
    return _POOLED(x, batch_actor)


def kernel(x, chunk_sizes, batch_actor):
    del chunk_sizes
    return _run(x, batch_actor)


# speedup vs baseline: 3.7183x; 2.3121x over previous
import functools

import jax
import jax.numpy as jnp
from jax import lax
from jax.experimental import pallas as pl
from jax.experimental.pallas import tpu as pltpu
from jax.experimental.pallas import tpu_sc as plsc

N = 320000
D = 128
NUM_CHUNKS = 8
NUM_SEG = 10000
CHUNK = N // NUM_CHUNKS
OUT_PER_CHUNK = NUM_SEG - 1

_INFO = plsc.get_sparse_core_info()
NC = _INFO.num_cores
NS = _INFO.num_subcores
WORKERS = NC * NS
PARTS = WORKERS // NUM_CHUNKS
SEG_PER_PART = NUM_SEG // PARTS
TILE_SEGS = 250
SUBTASKS = SEG_PER_PART // TILE_SEGS
BLK = 128
NVEC = D // 16


def _build():
    mesh = plsc.VectorSubcoreMesh(core_axis_name="c", subcore_axis_name="s")

    @functools.partial(
        pl.kernel,
        mesh=mesh,
        out_type=jax.ShapeDtypeStruct((NUM_CHUNKS * OUT_PER_CHUNK, D),
                                      jnp.float32),
        compiler_params=pltpu.CompilerParams(use_tc_tiling_on_sc=False, needs_layout_passes=False),
        scratch_types=[
            pltpu.VMEM((CHUNK + 16,), jnp.int32),
            pltpu.VMEM((TILE_SEGS, D), jnp.float32),
            pltpu.VMEM((TILE_SEGS, 16), jnp.float32),
            pltpu.VMEM((BLK, D), jnp.float32),
            pltpu.VMEM((BLK, D), jnp.float32),
            pltpu.SemaphoreType.DMA,
            pltpu.SemaphoreType.DMA,
        ],
    )
    def pooled(x_hbm, ids_hbm, out_hbm, ids_v, acc_v, cnt_v, xa_v, xb_v,
               sema, semb):
        wid = lax.axis_index("s") * NC + lax.axis_index("c")
        chunk = wid // PARTS
        part = wid % PARTS

        pltpu.sync_copy(ids_hbm.at[pl.ds(chunk * CHUNK, CHUNK)],
                        ids_v.at[pl.ds(0, CHUNK)])

        def idat(i):
            return ids_v[pl.ds(i, 16)][0]

        def ssorted(t):
            def body(_, st):
                lo, hi = st
                mid = (lo + hi) // 2
                v = idat(mid)
                lo2 = jnp.where(v < t, mid + 1, lo)
                hi2 = jnp.where(v < t, hi, mid)
                done = lo >= hi
                return (jnp.where(done, lo, lo2), jnp.where(done, hi, hi2))

            return lax.fori_loop(0, 16, body,
                                 (jnp.int32(0), jnp.int32(CHUNK)))[0]

        zero16 = jnp.zeros((16,), jnp.float32)
        one16 = jnp.ones((16,), jnp.float32)

        def qbody(q, _):
            lo_seg = part * SEG_PER_PART + q * TILE_SEGS
            rs = ssorted(lo_seg)
            re = ssorted(lo_seg + TILE_SEGS)

            def zbody(s, _):
                for v in range(NVEC):
                    acc_v[s, pl.ds(v * 16, 16)] = zero16
                cnt_v[s, :] = zero16
                return 0

            lax.fori_loop(0, TILE_SEGS, zbody, 0)

            kb0 = rs // BLK
            kb1 = (re + BLK - 1) // BLK
            nb = kb1 - kb0

            def issue(k, buf, sem):
                pltpu.make_async_copy(
                    x_hbm.at[pl.ds(chunk * CHUNK + k * BLK, BLK)],
                    buf, sem).start()

            def wait(buf, sem):
                pltpu.make_async_copy(
                    x_hbm.at[pl.ds(0, BLK)], buf, sem).wait()

            def row_add(buf, lid, ii):
                for v in range(NVEC):
                    sl = pl.ds(v * 16, 16)
                    plsc.addupdate(acc_v.at[lid, sl], buf[ii, sl])
                plsc.addupdate(cnt_v.at[lid], one16)

            def flush(lid, run, cntrun):
                for v in range(NVEC):
                    plsc.addupdate(acc_v.at[lid, pl.ds(v * 16, 16)], run[v])
                plsc.addupdate(cnt_v.at[lid], cntrun)

            def process(k, buf):
                def gbody(g, _):
                    rb = k * BLK + g * 16
                    i0g = jnp.maximum(rs, rb)
                    i1g = jnp.minimum(re, rb + 16)

                    @pl.when(i1g - i0g == 16)
                    def _():
                        ids16 = ids_v[pl.ds(rb, 16)]
                        lid = ids16[0] - lo_seg
                        run = [buf[g * 16, pl.ds(v * 16, 16)]
                               for v in range(NVEC)]
                        cntrun = one16
                        for j in range(1, 16):
                            lid_j = ids16[j] - lo_seg
                            same = lid_j == lid
                            same16 = jnp.full((16,), same)
                            run_prev = run
                            cnt_prev = cntrun

                            @pl.when(jnp.logical_not(same))
                            def _(lid=lid, run_prev=run_prev,
                                  cnt_prev=cnt_prev):
                                flush(lid, run_prev, cnt_prev)

                            xj = [buf[g * 16 + j, pl.ds(v * 16, 16)]
                                  for v in range(NVEC)]
                            run = [jnp.where(same16,
                                             run_prev[v] + xj[v], xj[v])
                                   for v in range(NVEC)]
                            cntrun = jnp.where(same16, cnt_prev + one16,
                                               one16)
                            lid = lid_j
                        flush(lid, run, cntrun)

                    @pl.when((i1g - i0g < 16) & (i1g > i0g))
                    def _():
                        def rbody(r, _):
                            row_add(buf, idat(r) - lo_seg, r - k * BLK)
                            return 0

                        lax.fori_loop(i0g, i1g, rbody, 0)

                    return 0

                lax.fori_loop(0, BLK // 16, gbody, 0)

            @pl.when(nb > 0)
            def _():
                issue(kb0, xa_v, sema)

            def tbody(t2, _):
                ke = kb0 + 2 * t2
                ko = ke + 1

                @pl.when(ke < kb1)
                def _():
                    @pl.when(ko < kb1)
                    def _():
                        issue(ko, xb_v, semb)

                    wait(xa_v, sema)
                    process(ke, xa_v)

                @pl.when(ko < kb1)
                def _():
                    @pl.when(ko + 1 < kb1)
                    def _():
                        issue(ko + 1, xa_v, sema)

                    wait(xb_v, semb)
                    process(ko, xb_v)

                return 0

            lax.fori_loop(0, (nb + 1) // 2, tbody, 0)

            def fbody(s, _):
                scale = 1.0 / jnp.maximum(cnt_v[s, :], 1.0)
                for v in range(NVEC):
                    sl = pl.ds(v * 16, 16)
                    acc_v[s, sl] = acc_v[s, sl] * scale
                return 0

            lax.fori_loop(0, TILE_SEGS, fbody, 0)

            out0 = chunk * OUT_PER_CHUNK

            @pl.when(lo_seg == 0)
            def _():
                pltpu.sync_copy(
                    acc_v.at[pl.ds(1, TILE_SEGS - 1)],
                    out_hbm.at[pl.ds(out0, TILE_SEGS - 1)])

            @pl.when(lo_seg > 0)
            def _():
                pltpu.sync_copy(
                    acc_v.at[pl.ds(0, TILE_SEGS)],
                    out_hbm.at[pl.ds(out0 + lo_seg - 1, TILE_SEGS)])

            return 0

        lax.fori_loop(0, SUBTASKS, qbody, 0)

    return pooled


_POOLED = _build()


@jax.jit
def _run(x, batch_actor):
    return _POOLED(x, batch_actor)


def kernel(x, chunk_sizes, batch_actor):
    del chunk_sizes
    return _run(x, batch_actor)
